# Initial kernel scaffold; baseline (speedup 1.0000x reference)
import jax
import jax.numpy as jnp
from jax.experimental import pallas as pl


def kernel(token_ids, edge_index, graph_ids, emb, W_self1, W_neigh1, b1, W_self2, W_neigh2, b2, Wc1, bc1, Wc2, bc2):
    raise NotImplementedError("write your pallas kernel here")


# speedup vs baseline: 4.0069x; 4.0069x over previous
import functools

import jax
import jax.numpy as jnp
from jax import lax
from jax.experimental import pallas as pl
from jax.experimental.pallas import tpu as pltpu
from jax.experimental.pallas import tpu_sc as plsc

N = 10000
E = 320000
D = 128
G = 64
NCLS = 10
NC, NS = 2, 16
NW = NC * NS
NPAD = 10240
RPT = NPAD // NW
RPS = NPAD // NS
C = 80
EPT = E // NW
NCHUNK = EPT // C

_mesh = plsc.VectorSubcoreMesh(core_axis_name="c", subcore_axis_name="s")


@functools.partial(
    pl.kernel,
    out_type=(
        jax.ShapeDtypeStruct((NPAD, D), jnp.float32),
        jax.ShapeDtypeStruct((NC, NPAD, D), jnp.float32),
    ),
    mesh=_mesh,
    scratch_types=[
        pltpu.VMEM((C,), jnp.int32),
        pltpu.VMEM((C, D), jnp.float32),
        pltpu.VMEM((C, D), jnp.float32),
        pltpu.VMEM_SHARED((NPAD, D), jnp.float32),
        pltpu.SemaphoreType.DMA,
    ],
)
def _embed_deg_k(tok_hbm, emb_hbm, dst_hbm, z_hbm, ones_hbm,
                 x_hbm, deg_hbm, idx_v, rows_v, ones_v, deg_sh, sem):
    cid = lax.axis_index("c")
    sid = lax.axis_index("s")
    wid = sid * NC + cid
    zoff = pl.multiple_of(sid * RPS, 8)
    pltpu.sync_copy(z_hbm.at[pl.ds(zoff, RPS)], deg_sh.at[pl.ds(zoff, RPS)])
    pltpu.sync_copy(ones_hbm, ones_v)
    plsc.subcore_barrier()

    ebase = wid * EPT

    def edge_body(i, _):
        off = pl.multiple_of(ebase + i * C, 8)
        pltpu.sync_copy(dst_hbm.at[pl.ds(off, C)], idx_v)
        pltpu.sync_copy(ones_v, deg_sh.at[idx_v], add=True)
        return 0

    lax.fori_loop(0, NCHUNK, edge_body, 0)

    rbase = wid * RPT

    def tok_body(i, _):
        off = pl.multiple_of(rbase + i * C, 8)
        pltpu.sync_copy(tok_hbm.at[pl.ds(off, C)], idx_v)
        pltpu.async_copy(emb_hbm.at[idx_v], rows_v, sem).wait()
        pltpu.sync_copy(rows_v, x_hbm.at[pl.ds(off, C)])
        return 0

    lax.fori_loop(0, RPT // C, tok_body, 0)

    plsc.subcore_barrier()
    pltpu.sync_copy(deg_sh.at[pl.ds(zoff, RPS)],
                    deg_hbm.at[cid, pl.ds(zoff, RPS)])


@functools.partial(
    pl.kernel,
    out_type=jax.ShapeDtypeStruct((NC, NPAD, D), jnp.float32),
    mesh=_mesh,
    scratch_types=[
        pltpu.VMEM((C,), jnp.int32),
        pltpu.VMEM((C,), jnp.int32),
        pltpu.VMEM((C, D), jnp.float32),
        pltpu.VMEM_SHARED((NPAD, D), jnp.float32),
        pltpu.SemaphoreType.DMA,
    ],
)
def _agg_k(src_hbm, dst_hbm, h_hbm, z_hbm,
           agg_hbm, src_v, dst_v, rows_v, agg_sh, sem):
    cid = lax.axis_index("c")
    sid = lax.axis_index("s")
    wid = sid * NC + cid
    zoff = pl.multiple_of(sid * RPS, 8)
    pltpu.sync_copy(z_hbm.at[pl.ds(zoff, RPS)], agg_sh.at[pl.ds(zoff, RPS)])
    plsc.subcore_barrier()

    ebase = wid * EPT

    def body(i, _):
        off = pl.multiple_of(ebase + i * C, 8)
        pltpu.sync_copy(src_hbm.at[pl.ds(off, C)], src_v)
        pltpu.sync_copy(dst_hbm.at[pl.ds(off, C)], dst_v)
        pltpu.async_copy(h_hbm.at[src_v], rows_v, sem).wait()
        pltpu.sync_copy(rows_v, agg_sh.at[dst_v], add=True)
        return 0

    lax.fori_loop(0, NCHUNK, body, 0)

    plsc.subcore_barrier()
    pltpu.sync_copy(agg_sh.at[pl.ds(zoff, RPS)],
                    agg_hbm.at[cid, pl.ds(zoff, RPS)])


_BLK = 256


def _sage_body(h_ref, a0_ref, a1_ref, d0_ref, d1_ref, ws_ref, wn_ref, b_ref,
               o_ref):
    deg = jnp.maximum(d0_ref[:, 0:1] + d1_ref[:, 0:1], 1.0)
    agg = (a0_ref[...] + a1_ref[...]) / deg
    acc = jnp.dot(h_ref[...], ws_ref[...], preferred_element_type=jnp.float32)
    acc += jnp.dot(agg, wn_ref[...], preferred_element_type=jnp.float32)
    o_ref[...] = jnp.maximum(acc + b_ref[...], 0.0)


def _sage(h, a0, a1, d0, d1, Ws, Wn, b):
    grid = (NPAD // _BLK,)
    return pl.pallas_call(
        _sage_body,
        grid=grid,
        in_specs=[
            pl.BlockSpec((_BLK, D), lambda i: (i, 0)),
            pl.BlockSpec((_BLK, D), lambda i: (i, 0)),
            pl.BlockSpec((_BLK, D), lambda i: (i, 0)),
            pl.BlockSpec((_BLK, D), lambda i: (i, 0)),
            pl.BlockSpec((_BLK, D), lambda i: (i, 0)),
            pl.BlockSpec((D, D), lambda i: (0, 0)),
            pl.BlockSpec((D, D), lambda i: (0, 0)),
            pl.BlockSpec((1, D), lambda i: (0, 0)),
        ],
        out_specs=pl.BlockSpec((_BLK, D), lambda i: (i, 0)),
        out_shape=jax.ShapeDtypeStruct((NPAD, D), jnp.float32),
    )(h, a0, a1, d0, d1, Ws, Wn, b)


def _cls_body(h_ref, gid_ref, wc1_ref, bc1_ref, wc2_ref, bc2_ref,
              o_ref, pooled_ref):
    def body(g, _):
        mask = gid_ref[...] == g
        vals = jnp.where(mask, h_ref[...], -jnp.inf)
        pooled_ref[pl.ds(g, 1), :] = jnp.max(vals, axis=0, keepdims=True)
        return 0

    lax.fori_loop(0, G, body, 0)
    pooled = pooled_ref[...]
    pooled = jnp.where(jnp.isfinite(pooled), pooled, 0.0)
    hid = jnp.dot(pooled, wc1_ref[...], preferred_element_type=jnp.float32)
    hid = jnp.maximum(hid + bc1_ref[...], 0.0)
    o_ref[...] = jnp.dot(hid, wc2_ref[...],
                         preferred_element_type=jnp.float32) + bc2_ref[...]


def _classifier(h2, gid, Wc1, bc1, Wc2p, bc2p):
    return pl.pallas_call(
        _cls_body,
        out_shape=jax.ShapeDtypeStruct((G, 128), jnp.float32),
        scratch_shapes=[pltpu.VMEM((G, D), jnp.float32)],
    )(h2, gid, Wc1, bc1, Wc2p, bc2p)


@jax.jit
def kernel(token_ids, edge_index, graph_ids, emb, W_self1, W_neigh1, b1,
           W_self2, W_neigh2, b2, Wc1, bc1, Wc2, bc2):
    tok = jnp.concatenate(
        [token_ids, jnp.zeros((NPAD - N,), jnp.int32)]).astype(jnp.int32)
    src = edge_index[0].astype(jnp.int32)
    dst = edge_index[1].astype(jnp.int32)
    gid = jnp.concatenate(
        [graph_ids, jnp.full((NPAD - N,), G, jnp.int32)]).reshape(NPAD, 1)
    z = jnp.zeros((NPAD, D), jnp.float32)
    ones = jnp.ones((C, D), jnp.float32)

    x, deg = _embed_deg_k(tok, emb, dst, z, ones)
    agg1 = _agg_k(src, dst, x, z)
    h1 = _sage(x, agg1[0], agg1[1], deg[0], deg[1],
               W_self1, W_neigh1, b1.reshape(1, D))
    agg2 = _agg_k(src, dst, h1, z)
    h2 = _sage(h1, agg2[0], agg2[1], deg[0], deg[1],
               W_self2, W_neigh2, b2.reshape(1, D))

    Wc2p = jnp.pad(Wc2, ((0, 0), (0, 128 - NCLS)))
    bc2p = jnp.pad(bc2, (0, 128 - NCLS)).reshape(1, 128)
    logits = _classifier(h2, gid, Wc1, bc1.reshape(1, D), Wc2p, bc2p)
    return logits[:, :NCLS]
